# Initial kernel scaffold; baseline (speedup 1.0000x reference)
import jax
import jax.numpy as jnp
from jax.experimental import pallas as pl


def kernel(x, edge_index, batch, W1, b1, W2, b2):
    raise NotImplementedError("write your pallas kernel here")


# speedup vs baseline: 19.1908x; 19.1908x over previous
import functools

import jax
import jax.numpy as jnp
from jax import lax
from jax.experimental import pallas as pl
from jax.experimental.pallas import tpu as pltpu
from jax.experimental.pallas import tpu_sc as plsc

N = 10000
E = 320000
D_IN, D_H, D_OUT = 128, 64, 16

NC, NS = 2, 16
NW = NC * NS
CHUNK = 128
KCH = 8
EPW = 10240
E_PAD = NW * EPW
GROUPS = EPW // (CHUNK * KCH)
EROWS_PW = EPW // CHUNK
ROWS_PAD = 10240
R_INIT = 624
R_TAIL = N - NS * R_INIT
R_OUT = ROWS_PAD // NS
RB = 1000


def _sc_mesh():
    return plsc.VectorSubcoreMesh(core_axis_name="c", subcore_axis_name="s")


DW = 16

@functools.partial(
    pl.kernel,
    out_type=jax.ShapeDtypeStruct((NC, ROWS_PAD, DW), jnp.float32),
    mesh=_sc_mesh(),
    scratch_types=[
        pltpu.VMEM((KCH, CHUNK), jnp.int32),
        pltpu.VMEM((CHUNK, DW), jnp.float32),
        pltpu.VMEM_SHARED((ROWS_PAD, DW), jnp.float32),
        pltpu.SemaphoreType.DMA,
    ],
    compiler_params=pltpu.CompilerParams(use_tc_tiling_on_sc=False),
)
def _deg_sc(zero_hbm, ones_hbm, dst_hbm, out_hbm, didx, ones_v, acc, ssem):
    cid = lax.axis_index("c")
    sid = lax.axis_index("s")
    wid = sid * NC + cid
    pltpu.sync_copy(zero_hbm.at[pl.ds(sid * R_OUT, R_OUT)],
                    acc.at[pl.ds(sid * R_OUT, R_OUT)])
    pltpu.sync_copy(ones_hbm, ones_v)
    plsc.subcore_barrier()

    def group(g, carry):
        row0 = wid * EROWS_PW + g * KCH
        pltpu.sync_copy(dst_hbm.at[pl.ds(row0, KCH)], didx)
        descs = [pltpu.async_copy(ones_v, acc.at[didx.at[k]], ssem, add=True)
                 for k in range(KCH)]
        for d in descs:
            d.wait()
        return carry

    lax.fori_loop(0, GROUPS, group, 0)
    plsc.subcore_barrier()
    pltpu.sync_copy(acc.at[pl.ds(sid * R_OUT, R_OUT)],
                    out_hbm.at[cid, pl.ds(sid * R_OUT, R_OUT)])


def _make_agg(d):
    @functools.partial(
        pl.kernel,
        out_type=jax.ShapeDtypeStruct((NC, ROWS_PAD, d), jnp.float32),
        mesh=_sc_mesh(),
        scratch_types=[
            pltpu.VMEM((KCH, CHUNK), jnp.int32),
            pltpu.VMEM((KCH, CHUNK), jnp.int32),
            pltpu.VMEM((KCH, CHUNK, d), jnp.float32),
            pltpu.VMEM_SHARED((ROWS_PAD, d), jnp.float32),
            pltpu.SemaphoreType.DMA,
            pltpu.SemaphoreType.DMA,
        ],
        compiler_params=pltpu.CompilerParams(use_tc_tiling_on_sc=False),
    )
    def agg(y_hbm, zeros_hbm, src_hbm, dst_hbm, out_hbm,
            sidx, didx, rows, acc, gsem, ssem):
        cid = lax.axis_index("c")
        sid = lax.axis_index("s")
        wid = sid * NC + cid

        @pl.when(cid == 0)
        def _():
            pltpu.sync_copy(y_hbm.at[pl.ds(sid * R_INIT, R_INIT)],
                            acc.at[pl.ds(sid * R_INIT, R_INIT)])

            @pl.when(sid == NS - 1)
            def _():
                pltpu.sync_copy(y_hbm.at[pl.ds(NS * R_INIT, R_TAIL)],
                                acc.at[pl.ds(NS * R_INIT, R_TAIL)])

        @pl.when(cid != 0)
        def _():
            pltpu.sync_copy(zeros_hbm.at[pl.ds(sid * R_INIT, R_INIT)],
                            acc.at[pl.ds(sid * R_INIT, R_INIT)])

            @pl.when(sid == NS - 1)
            def _():
                pltpu.sync_copy(zeros_hbm.at[pl.ds(NS * R_INIT, R_TAIL)],
                                acc.at[pl.ds(NS * R_INIT, R_TAIL)])

        plsc.subcore_barrier()

        def group(g, carry):
            row0 = wid * EROWS_PW + g * KCH
            pltpu.sync_copy(src_hbm.at[pl.ds(row0, KCH)], sidx)
            pltpu.sync_copy(dst_hbm.at[pl.ds(row0, KCH)], didx)
            gds = [pltpu.async_copy(y_hbm.at[sidx.at[k]], rows.at[k], gsem)
                   for k in range(KCH)]
            for dsc in gds:
                dsc.wait()
            sds = [pltpu.async_copy(rows.at[k], acc.at[didx.at[k]], ssem,
                                    add=True)
                   for k in range(KCH)]
            for dsc in sds:
                dsc.wait()
            return carry

        lax.fori_loop(0, GROUPS, group, 0)
        plsc.subcore_barrier()
        pltpu.sync_copy(acc.at[pl.ds(sid * R_OUT, R_OUT)],
                        out_hbm.at[cid, pl.ds(sid * R_OUT, R_OUT)])

    return agg


_agg64 = _make_agg(D_H)
_agg16 = _make_agg(D_OUT)


def _tc1(x, W1, degt):
    def body(x_ref, w_ref, deg_ref, y_ref, dinv_ref):
        d2 = deg_ref[...]
        deg = d2[:, 0:1] + d2[:, 1:2] + 1.0
        dinv = lax.rsqrt(deg)
        xw = jnp.dot(x_ref[...], w_ref[...], preferred_element_type=jnp.float32)
        y_ref[...] = xw * dinv
        dinv_ref[...] = dinv

    return pl.pallas_call(
        body,
        grid=(N // RB,),
        in_specs=[
            pl.BlockSpec((RB, D_IN), lambda i: (i, 0)),
            pl.BlockSpec((D_IN, D_H), lambda i: (0, 0)),
            pl.BlockSpec((RB, NC), lambda i: (i, 0)),
        ],
        out_specs=[
            pl.BlockSpec((RB, D_H), lambda i: (i, 0)),
            pl.BlockSpec((RB, 1), lambda i: (i, 0)),
        ],
        out_shape=[
            jax.ShapeDtypeStruct((N, D_H), jnp.float32),
            jax.ShapeDtypeStruct((N, 1), jnp.float32),
        ],
    )(x, W1, degt)


def _tc2(p, dinv, b1, W2):
    def body(p_ref, dinv_ref, b1_ref, w2_ref, y2_ref):
        s = p_ref[0] + p_ref[1]
        h = jnp.maximum(s * dinv_ref[...] + b1_ref[...], 0.0)
        y2 = jnp.dot(h, w2_ref[...], preferred_element_type=jnp.float32)
        y2_ref[...] = y2 * dinv_ref[...]

    return pl.pallas_call(
        body,
        grid=(N // RB,),
        in_specs=[
            pl.BlockSpec((NC, RB, D_H), lambda i: (0, i, 0)),
            pl.BlockSpec((RB, 1), lambda i: (i, 0)),
            pl.BlockSpec((1, D_H), lambda i: (0, 0)),
            pl.BlockSpec((D_H, D_OUT), lambda i: (0, 0)),
        ],
        out_specs=pl.BlockSpec((RB, D_OUT), lambda i: (i, 0)),
        out_shape=jax.ShapeDtypeStruct((N, D_OUT), jnp.float32),
    )(p, dinv, b1, W2)


def _tc3(q, dinv, b2):
    nblk = N // RB

    def body(q_ref, dinv_ref, b2_ref, out_ref, sum_ref):
        i = pl.program_id(0)
        o = (q_ref[0] + q_ref[1]) * dinv_ref[...] + b2_ref[...]
        out_ref[...] = o

        @pl.when(i == 0)
        def _():
            sum_ref[...] = jnp.zeros_like(sum_ref)

        sum_ref[...] += jnp.sum(o, axis=0, keepdims=True)

        @pl.when(i == nblk - 1)
        def _():
            sum_ref[...] = sum_ref[...] * (1.0 / N)

    return pl.pallas_call(
        body,
        grid=(nblk,),
        in_specs=[
            pl.BlockSpec((NC, RB, D_OUT), lambda i: (0, i, 0)),
            pl.BlockSpec((RB, 1), lambda i: (i, 0)),
            pl.BlockSpec((1, D_OUT), lambda i: (0, 0)),
        ],
        out_specs=[
            pl.BlockSpec((RB, D_OUT), lambda i: (i, 0)),
            pl.BlockSpec((1, D_OUT), lambda i: (0, 0)),
        ],
        out_shape=[
            jax.ShapeDtypeStruct((N, D_OUT), jnp.float32),
            jax.ShapeDtypeStruct((1, D_OUT), jnp.float32),
        ],
    )(q, dinv, b2)


def kernel(x, edge_index, batch, W1, b1, W2, b2):
    src = edge_index[0]
    dst = edge_index[1]
    src_p = jnp.concatenate([src, jnp.zeros((E_PAD - E,), jnp.int32)])
    dst_p = jnp.concatenate([dst, jnp.full((E_PAD - E,), N, jnp.int32)])
    src2 = src_p.reshape(E_PAD // CHUNK, CHUNK)
    dst2 = dst_p.reshape(E_PAD // CHUNK, CHUNK)

    zeros_deg = jnp.zeros((ROWS_PAD, DW), jnp.float32)
    ones_col = jnp.ones((CHUNK, DW), jnp.float32)
    zeros_h = jnp.zeros((N, D_H), jnp.float32)
    zeros_o = jnp.zeros((N, D_OUT), jnp.float32)

    degp = _deg_sc(zeros_deg, ones_col, dst2)
    degt = degp[:, :, 0].T

    y1, dinv = _tc1(x, W1, degt)
    p = _agg64(y1, zeros_h, src2, dst2)
    y2 = _tc2(p, dinv, b1.reshape(1, D_H), W2)
    q = _agg16(y2, zeros_o, src2, dst2)
    out, gsum = _tc3(q, dinv, b2.reshape(1, D_OUT))
    return (out, gsum)


# speedup vs baseline: 21.0350x; 1.0961x over previous
import functools

import jax
import jax.numpy as jnp
from jax import lax
from jax.experimental import pallas as pl
from jax.experimental.pallas import tpu as pltpu
from jax.experimental.pallas import tpu_sc as plsc

N = 10000
E = 320000
D_IN, D_H, D_OUT = 128, 64, 16

NC, NS = 2, 16
NW = NC * NS
CHUNK = 128
KCH = 8
EPW = 10240
E_PAD = NW * EPW
GROUPS = EPW // (CHUNK * KCH)
EROWS_PW = EPW // CHUNK
ROWS_PAD = 10240
R_INIT = 624
R_TAIL = N - NS * R_INIT
R_OUT = ROWS_PAD // NS
RB = 1000


def _sc_mesh():
    return plsc.VectorSubcoreMesh(core_axis_name="c", subcore_axis_name="s")


DW = 16

@functools.partial(
    pl.kernel,
    out_type=jax.ShapeDtypeStruct((NC, ROWS_PAD, DW), jnp.float32),
    mesh=_sc_mesh(),
    scratch_types=[
        pltpu.VMEM((EROWS_PW, CHUNK), jnp.int32),
        pltpu.VMEM((CHUNK, DW), jnp.float32),
        pltpu.VMEM_SHARED((ROWS_PAD, DW), jnp.float32),
        pltpu.SemaphoreType.DMA,
    ],
    compiler_params=pltpu.CompilerParams(use_tc_tiling_on_sc=False),
)
def _deg_sc(zero_hbm, ones_hbm, dst_hbm, out_hbm, didx, ones_v, acc, ssem):
    cid = lax.axis_index("c")
    sid = lax.axis_index("s")
    wid = sid * NC + cid
    pltpu.sync_copy(zero_hbm.at[pl.ds(sid * R_OUT, R_OUT)],
                    acc.at[pl.ds(sid * R_OUT, R_OUT)])
    pltpu.sync_copy(ones_hbm, ones_v)
    plsc.subcore_barrier()

    pltpu.sync_copy(dst_hbm.at[pl.ds(wid * EROWS_PW, EROWS_PW)], didx)

    def fire(g):
        for k in range(KCH):
            pltpu.async_copy(ones_v, acc.at[didx.at[g * KCH + k]], ssem,
                             add=True)

    def drain(g):
        for k in range(KCH):
            pltpu.make_async_copy(ones_v, acc.at[didx.at[g * KCH + k]],
                                  ssem).wait()

    fire(0)

    def group(g, carry):
        fire(g)
        drain(g - 1)
        return carry

    lax.fori_loop(1, GROUPS, group, 0)
    drain(GROUPS - 1)
    plsc.subcore_barrier()
    pltpu.sync_copy(acc.at[pl.ds(sid * R_OUT, R_OUT)],
                    out_hbm.at[cid, pl.ds(sid * R_OUT, R_OUT)])


def _make_agg(d, kch):
    groups = EROWS_PW // kch

    @functools.partial(
        pl.kernel,
        out_type=jax.ShapeDtypeStruct((NC, ROWS_PAD, d), jnp.float32),
        mesh=_sc_mesh(),
        scratch_types=[
            pltpu.VMEM((EROWS_PW, CHUNK), jnp.int32),
            pltpu.VMEM((EROWS_PW, CHUNK), jnp.int32),
            pltpu.VMEM((2, kch, CHUNK, d), jnp.float32),
            pltpu.VMEM_SHARED((ROWS_PAD, d), jnp.float32),
            pltpu.SemaphoreType.DMA,
            pltpu.SemaphoreType.DMA,
        ],
        compiler_params=pltpu.CompilerParams(use_tc_tiling_on_sc=False),
    )
    def agg(y_hbm, zeros_hbm, src_hbm, dst_hbm, out_hbm,
            sidx, didx, rows, acc, gsem, ssem):
        cid = lax.axis_index("c")
        sid = lax.axis_index("s")
        wid = sid * NC + cid

        @pl.when(cid == 0)
        def _():
            pltpu.sync_copy(y_hbm.at[pl.ds(sid * R_INIT, R_INIT)],
                            acc.at[pl.ds(sid * R_INIT, R_INIT)])

            @pl.when(sid == NS - 1)
            def _():
                pltpu.sync_copy(y_hbm.at[pl.ds(NS * R_INIT, R_TAIL)],
                                acc.at[pl.ds(NS * R_INIT, R_TAIL)])

        @pl.when(cid != 0)
        def _():
            pltpu.sync_copy(zeros_hbm.at[pl.ds(sid * R_INIT, R_INIT)],
                            acc.at[pl.ds(sid * R_INIT, R_INIT)])

            @pl.when(sid == NS - 1)
            def _():
                pltpu.sync_copy(zeros_hbm.at[pl.ds(NS * R_INIT, R_TAIL)],
                                acc.at[pl.ds(NS * R_INIT, R_TAIL)])

        row_base = wid * EROWS_PW
        pltpu.sync_copy(src_hbm.at[pl.ds(row_base, EROWS_PW)], sidx)
        pltpu.sync_copy(dst_hbm.at[pl.ds(row_base, EROWS_PW)], didx)
        plsc.subcore_barrier()

        def fire_g(g, buf):
            for k in range(kch):
                pltpu.async_copy(y_hbm.at[sidx.at[g * kch + k]],
                                 rows.at[buf, k], gsem)

        def wait_g(g, buf):
            for k in range(kch):
                pltpu.make_async_copy(y_hbm.at[sidx.at[g * kch + k]],
                                      rows.at[buf, k], gsem).wait()

        def fire_s(g, buf):
            for k in range(kch):
                pltpu.async_copy(rows.at[buf, k], acc.at[didx.at[g * kch + k]],
                                 ssem, add=True)

        def wait_s(g, buf):
            for k in range(kch):
                pltpu.make_async_copy(rows.at[buf, k],
                                      acc.at[didx.at[g * kch + k]],
                                      ssem).wait()

        fire_g(0, 0)
        fire_g(1, 1)
        wait_g(0, 0)
        fire_s(0, 0)

        def group(g, carry):
            bc = g % 2
            bp = (g - 1) % 2
            wait_s(g - 2, bc)
            fire_g(g, bc)
            wait_g(g - 1, bp)
            fire_s(g - 1, bp)
            return carry

        lax.fori_loop(2, groups, group, 0)
        wait_g(groups - 1, (groups - 1) % 2)
        wait_s(groups - 2, (groups - 2) % 2)
        fire_s(groups - 1, (groups - 1) % 2)
        wait_s(groups - 1, (groups - 1) % 2)
        plsc.subcore_barrier()
        pltpu.sync_copy(acc.at[pl.ds(sid * R_OUT, R_OUT)],
                        out_hbm.at[cid, pl.ds(sid * R_OUT, R_OUT)])

    return agg


_agg64 = _make_agg(D_H, 4)
_agg16 = _make_agg(D_OUT, 8)


def _tc1(x, W1, degt):
    def body(x_ref, w_ref, deg_ref, y_ref, dinv_ref):
        d2 = deg_ref[...]
        deg = d2[:, 0:1] + d2[:, 1:2] + 1.0
        dinv = lax.rsqrt(deg)
        xw = jnp.dot(x_ref[...], w_ref[...], preferred_element_type=jnp.float32)
        y_ref[...] = xw * dinv
        dinv_ref[...] = dinv

    return pl.pallas_call(
        body,
        grid=(N // RB,),
        in_specs=[
            pl.BlockSpec((RB, D_IN), lambda i: (i, 0)),
            pl.BlockSpec((D_IN, D_H), lambda i: (0, 0)),
            pl.BlockSpec((RB, NC), lambda i: (i, 0)),
        ],
        out_specs=[
            pl.BlockSpec((RB, D_H), lambda i: (i, 0)),
            pl.BlockSpec((RB, 1), lambda i: (i, 0)),
        ],
        out_shape=[
            jax.ShapeDtypeStruct((N, D_H), jnp.float32),
            jax.ShapeDtypeStruct((N, 1), jnp.float32),
        ],
    )(x, W1, degt)


def _tc2(p, dinv, b1, W2):
    def body(p_ref, dinv_ref, b1_ref, w2_ref, y2_ref):
        s = p_ref[0] + p_ref[1]
        h = jnp.maximum(s * dinv_ref[...] + b1_ref[...], 0.0)
        y2 = jnp.dot(h, w2_ref[...], preferred_element_type=jnp.float32)
        y2_ref[...] = y2 * dinv_ref[...]

    return pl.pallas_call(
        body,
        grid=(N // RB,),
        in_specs=[
            pl.BlockSpec((NC, RB, D_H), lambda i: (0, i, 0)),
            pl.BlockSpec((RB, 1), lambda i: (i, 0)),
            pl.BlockSpec((1, D_H), lambda i: (0, 0)),
            pl.BlockSpec((D_H, D_OUT), lambda i: (0, 0)),
        ],
        out_specs=pl.BlockSpec((RB, D_OUT), lambda i: (i, 0)),
        out_shape=jax.ShapeDtypeStruct((N, D_OUT), jnp.float32),
    )(p, dinv, b1, W2)


def _tc3(q, dinv, b2):
    nblk = N // RB

    def body(q_ref, dinv_ref, b2_ref, out_ref, sum_ref):
        i = pl.program_id(0)
        o = (q_ref[0] + q_ref[1]) * dinv_ref[...] + b2_ref[...]
        out_ref[...] = o

        @pl.when(i == 0)
        def _():
            sum_ref[...] = jnp.zeros_like(sum_ref)

        sum_ref[...] += jnp.sum(o, axis=0, keepdims=True)

        @pl.when(i == nblk - 1)
        def _():
            sum_ref[...] = sum_ref[...] * (1.0 / N)

    return pl.pallas_call(
        body,
        grid=(nblk,),
        in_specs=[
            pl.BlockSpec((NC, RB, D_OUT), lambda i: (0, i, 0)),
            pl.BlockSpec((RB, 1), lambda i: (i, 0)),
            pl.BlockSpec((1, D_OUT), lambda i: (0, 0)),
        ],
        out_specs=[
            pl.BlockSpec((RB, D_OUT), lambda i: (i, 0)),
            pl.BlockSpec((1, D_OUT), lambda i: (0, 0)),
        ],
        out_shape=[
            jax.ShapeDtypeStruct((N, D_OUT), jnp.float32),
            jax.ShapeDtypeStruct((1, D_OUT), jnp.float32),
        ],
    )(q, dinv, b2)


def kernel(x, edge_index, batch, W1, b1, W2, b2):
    src = edge_index[0]
    dst = edge_index[1]
    src_p = jnp.concatenate([src, jnp.zeros((E_PAD - E,), jnp.int32)])
    dst_p = jnp.concatenate([dst, jnp.full((E_PAD - E,), N, jnp.int32)])
    src2 = src_p.reshape(E_PAD // CHUNK, CHUNK)
    dst2 = dst_p.reshape(E_PAD // CHUNK, CHUNK)

    zeros_deg = jnp.zeros((ROWS_PAD, DW), jnp.float32)
    ones_col = jnp.ones((CHUNK, DW), jnp.float32)
    zeros_h = jnp.zeros((N, D_H), jnp.float32)
    zeros_o = jnp.zeros((N, D_OUT), jnp.float32)

    degp = _deg_sc(zeros_deg, ones_col, dst2)
    degt = degp[:, :, 0].T

    y1, dinv = _tc1(x, W1, degt)
    p = _agg64(y1, zeros_h, src2, dst2)
    y2 = _tc2(p, dinv, b1.reshape(1, D_H), W2)
    q = _agg16(y2, zeros_o, src2, dst2)
    out, gsum = _tc3(q, dinv, b2.reshape(1, D_OUT))
    return (out, gsum)
